# baseline (speedup 1.0000x reference)
import functools

import jax
import jax.numpy as jnp
from jax import lax
from jax.experimental import pallas as pl
from jax.experimental.pallas import tpu as pltpu
from jax.experimental.pallas import tpu_sc as plsc

B = 1024
D = 256
GEN_VOC = 50000
OUT_VOC = 100000
L = 200

GPAD = 50176
BN = 1024
J = GPAD // BN
BB = 1024
JPAD = 128
LPAD = 256
OPAD = 100096

NEG = -1e30


def _tc_body(x_ref, w_ref, b_ref, sp_ref, sraw_ref, corr_ref, ptr_ref,
             m_scr, l_scr, mh_scr):
    j = pl.program_id(1)
    BB = x_ref.shape[0]
    s = jnp.dot(x_ref[...], w_ref[...], preferred_element_type=jnp.float32)
    s = s + b_ref[...]

    col = j * BN + lax.broadcasted_iota(jnp.int32, s.shape, 1)
    sm = jnp.where(col < GEN_VOC, s, NEG)

    @pl.when(j == 0)
    def _():
        m_scr[...] = jnp.full((BB, 1), NEG, jnp.float32)
        l_scr[...] = jnp.zeros((BB, 1), jnp.float32)
        mh_scr[...] = jnp.full((BB, JPAD), NEG, jnp.float32)

    m_old = m_scr[...]
    l_old = l_scr[...]
    pm = jnp.max(sm, axis=1, keepdims=True)
    m_new = jnp.maximum(m_old, pm)
    l_new = l_old * jnp.exp(m_old - m_new) + jnp.sum(
        jnp.exp(sm - m_new), axis=1, keepdims=True)
    m_scr[...] = m_new
    l_scr[...] = l_new
    jcol = lax.broadcasted_iota(jnp.int32, (BB, JPAD), 1)
    mh_scr[...] = jnp.where(jcol == j, m_new, mh_scr[...])

    sraw_ref[...] = jnp.exp(sm - m_new).reshape(BB, BN // 128, 128)

    @pl.when(j == J - 1)
    def _():
        sp = sp_ref[...]
        pm2 = jnp.max(sp, axis=1, keepdims=True)
        m_f = jnp.maximum(m_new, pm2)
        l_f = l_new * jnp.exp(m_new - m_f) + jnp.sum(
            jnp.exp(sp - m_f), axis=1, keepdims=True)
        ptr_ref[...] = jnp.exp(sp - m_f) / l_f
        corr_ref[...] = jnp.exp(mh_scr[...] - m_f) / l_f


def _tc_stage(x, w_pad, b_pad, sp):
    NB = x.shape[0]
    BB = NB
    return pl.pallas_call(
        _tc_body,
        grid=(NB // BB, J),
        in_specs=[
            pl.BlockSpec((BB, D), lambda i, j: (i, 0)),
            pl.BlockSpec((D, BN), lambda i, j: (0, j)),
            pl.BlockSpec((1, BN), lambda i, j: (0, j)),
            pl.BlockSpec((BB, LPAD), lambda i, j: (i, 0)),
        ],
        out_specs=[
            pl.BlockSpec((BB, BN // 128, 128), lambda i, j: (i, j, 0)),
            pl.BlockSpec((BB, JPAD), lambda i, j: (i, 0)),
            pl.BlockSpec((BB, LPAD), lambda i, j: (i, 0)),
        ],
        out_shape=[
            jax.ShapeDtypeStruct((NB, GPAD // 128, 128), jnp.float32),
            jax.ShapeDtypeStruct((NB, JPAD), jnp.float32),
            jax.ShapeDtypeStruct((NB, LPAD), jnp.float32),
        ],
        scratch_shapes=[
            pltpu.VMEM((BB, 1), jnp.float32),
            pltpu.VMEM((BB, 1), jnp.float32),
            pltpu.VMEM((BB, JPAD), jnp.float32),
        ],
        compiler_params=pltpu.CompilerParams(
            dimension_semantics=("arbitrary", "arbitrary")),
    )(x, w_pad, b_pad, sp)


NW = 32
GRP = 4
CHP = 56
NPC = (GPAD // 128) // CHP
CHI = 3584
NIC = GPAD // CHI
IPP = (CHP * 128) // CHI
NVI = CHI // 16
SEC = OPAD // 4


def _sc_scatter(probs, corr, ptr, ctx, gen_to_out, inp_to_out):
    NB = ptr.shape[0]
    RPT = NB // NW
    mesh = plsc.VectorSubcoreMesh(core_axis_name="c", subcore_axis_name="s")

    @functools.partial(
        pl.kernel,
        out_type=jax.ShapeDtypeStruct((NB * OPAD,), jnp.float32),
        mesh=mesh,
        scratch_types=[
            pltpu.VMEM((OPAD,), jnp.float32),
            pltpu.VMEM((2, CHP, 128), jnp.float32),
            pltpu.VMEM((2, CHI), jnp.int32),
            pltpu.VMEM((GRP, JPAD), jnp.float32),
            pltpu.VMEM((GRP, LPAD), jnp.float32),
            pltpu.VMEM((GRP, LPAD), jnp.int32),
            pltpu.VMEM((13, 16), jnp.int32),
            pltpu.SemaphoreType.DMA,
            pltpu.SemaphoreType.DMA,
            pltpu.SemaphoreType.DMA,
            pltpu.SemaphoreType.DMA,
            pltpu.SemaphoreType.DMA,
            pltpu.SemaphoreType.DMA,
        ],
        compiler_params=pltpu.CompilerParams(needs_layout_passes=False),
    )
    def k(sraw_h, corr_h, ptr_h, ctx_h, gidx_h, ito_h, out_h,
          acc, pbuf, ibuf, corrbuf, ptrbuf, ctxbuf, ctxobuf,
          semp0, semp1, semi0, semi1, sem_c, sem_o):
        cid = lax.axis_index("c")
        sid = lax.axis_index("s")
        wid = sid * 2 + cid
        base = wid * RPT
        semp = (semp0, semp1)
        semi = (semi0, semi1)

        def zero_sec(s4):
            def zbody(i, carry):
                acc[pl.ds(s4 * SEC + i * 16, 16)] = jnp.zeros(
                    (16,), jnp.float32)
                return carry
            lax.fori_loop(0, SEC // 16, zbody, 0, unroll=16)

        for s4 in range(4):
            zero_sec(s4)

        def pchunk_copy(row, pc, bufsel):
            return pltpu.make_async_copy(
                sraw_h.at[row, pl.ds(pc * CHP, CHP)], pbuf.at[bufsel],
                semp[bufsel])

        def ichunk_copy(ic, bufsel):
            return pltpu.make_async_copy(
                gidx_h.at[pl.ds(ic * CHI, CHI)], ibuf.at[bufsel],
                semi[bufsel])

        def prime(row):
            for w in range(2):
                pchunk_copy(row, w, w).start()
                ichunk_copy(w, w).start()

        prime(base)

        def grp_body(g, carry):
            gbase = base + g * GRP
            pltpu.sync_copy(corr_h.at[pl.ds(gbase, GRP)], corrbuf)
            pltpu.sync_copy(ptr_h.at[pl.ds(gbase, GRP)], ptrbuf)
            pltpu.sync_copy(ctx_h.at[pl.ds(gbase, GRP)], ctxbuf)

            def row_body(rr, carry):
                row = gbase + rr
                gath = []
                for kk in range(13):
                    h = pltpu.make_async_copy(
                        ito_h.at[ctxbuf.at[rr, pl.ds(kk * 16, 16)]],
                        ctxobuf.at[kk], sem_c)
                    h.start()
                    gath.append(h)

                for pc in range(NPC):
                    pp = pc % 2
                    pchunk_copy(row, pc, pp).wait()
                    for ici in range(IPP):
                        ic = pc * IPP + ici
                        pi = ic % 2
                        ichunk_copy(ic, pi).wait()

                        def scat(kv, carry):
                            iv = ibuf[pi, pl.ds(kv * 16, 16)]
                            r = ici * (CHI // 128) + kv // 8
                            c = pl.multiple_of((kv % 8) * 16, 16)
                            pv = pbuf[pp, r, pl.ds(c, 16)]
                            jb = (ic * NVI + kv) // (BN // 16)
                            cv = plsc.load_gather(
                                corrbuf,
                                [jnp.full((16,), rr, jnp.int32),
                                 jnp.full((16,), jb, jnp.int32)])
                            plsc.addupdate_scatter(acc, [iv], pv * cv)
                            return carry

                        lax.fori_loop(0, NVI, scat, 0, unroll=16)
                        if ic + 2 < NIC:
                            ichunk_copy(ic + 2, pi).start()
                    if pc + 2 < NPC:
                        pchunk_copy(row, pc + 2, pp).start()

                for kk in range(13):
                    gath[kk].wait()
                for kk in range(13):
                    pv = ptrbuf[rr, pl.ds(kk * 16, 16)]
                    plsc.addupdate_scatter(acc, [ctxobuf[kk]], pv)

                @pl.when(row + 1 < base + RPT)
                def _():
                    prime(row + 1)

                secs = []
                for s4 in range(4):
                    h = pltpu.make_async_copy(
                        acc.at[pl.ds(s4 * SEC, SEC)],
                        out_h.at[pl.ds(row * OPAD + s4 * SEC, SEC)],
                        sem_o)
                    h.start()
                    secs.append(h)
                for s4 in range(4):
                    secs[s4].wait()
                    zero_sec(s4)
                return carry

            lax.fori_loop(0, GRP, row_body, 0)
            return carry

        lax.fori_loop(0, RPT // GRP, grp_body, 0)

    return k(probs, corr, ptr, ctx, gen_to_out, inp_to_out)


def kernel(x, scores, ctx_inp, W_gen, b_gen, gen_to_out, inp_to_out):
    w_pad = jnp.pad(W_gen, ((0, 0), (0, GPAD - GEN_VOC))).astype(jnp.bfloat16)
    b_pad = jnp.pad(b_gen, (0, GPAD - GEN_VOC)).reshape(1, GPAD)
    sp = jnp.pad(scores.astype(jnp.float32), ((0, 0), (0, LPAD - L)),
                 constant_values=NEG)
    ctxp = jnp.pad(ctx_inp.astype(jnp.int32), ((0, 0), (0, LPAD - L)))
    gidx = jnp.pad(gen_to_out.astype(jnp.int32), (0, GPAD - GEN_VOC))
    ito = inp_to_out.astype(jnp.int32)
    xb = x.astype(jnp.bfloat16)

    NH = 2
    NBH = B // NH
    outs = []
    for h in range(NH):
        rs = slice(h * NBH, (h + 1) * NBH)
        probs, corr, ptr = _tc_stage(xb[rs], w_pad, b_pad, sp[rs])
        out_flat = _sc_scatter(probs, corr, ptr, ctxp[rs], gidx, ito)
        outs.append(out_flat.reshape(NBH, OPAD)[:, :OUT_VOC])
    return jnp.concatenate(outs, axis=0)


# speedup vs baseline: 1.0784x; 1.0784x over previous
import functools

import jax
import jax.numpy as jnp
from jax import lax
from jax.experimental import pallas as pl
from jax.experimental.pallas import tpu as pltpu
from jax.experimental.pallas import tpu_sc as plsc

B = 1024
D = 256
GEN_VOC = 50000
OUT_VOC = 100000
L = 200

GPAD = 50176
BN = 1024
J = GPAD // BN
BB = 1024
JPAD = 128
LPAD = 256
OPAD = 100096

NEG = -1e30


def _tc_body(x_ref, w_ref, b_ref, sp_ref, sraw_ref, corr_ref, ptr_ref,
             m_scr, l_scr, mh_scr):
    j = pl.program_id(1)
    BB = x_ref.shape[0]
    s = jnp.dot(x_ref[...], w_ref[...], preferred_element_type=jnp.float32)
    s = s + b_ref[...]

    col = j * BN + lax.broadcasted_iota(jnp.int32, s.shape, 1)
    sm = jnp.where(col < GEN_VOC, s, NEG)

    @pl.when(j == 0)
    def _():
        m_scr[...] = jnp.full((BB, 1), NEG, jnp.float32)
        l_scr[...] = jnp.zeros((BB, 1), jnp.float32)
        mh_scr[...] = jnp.full((BB, JPAD), NEG, jnp.float32)

    m_old = m_scr[...]
    l_old = l_scr[...]
    pm = jnp.max(sm, axis=1, keepdims=True)
    m_new = jnp.maximum(m_old, pm)
    l_new = l_old * jnp.exp(m_old - m_new) + jnp.sum(
        jnp.exp(sm - m_new), axis=1, keepdims=True)
    m_scr[...] = m_new
    l_scr[...] = l_new
    jcol = lax.broadcasted_iota(jnp.int32, (BB, JPAD), 1)
    mh_scr[...] = jnp.where(jcol == j, m_new, mh_scr[...])

    sraw_ref[...] = jnp.exp(sm - m_new).reshape(BB, BN // 128, 128)

    @pl.when(j == J - 1)
    def _():
        sp = sp_ref[...]
        pm2 = jnp.max(sp, axis=1, keepdims=True)
        m_f = jnp.maximum(m_new, pm2)
        l_f = l_new * jnp.exp(m_new - m_f) + jnp.sum(
            jnp.exp(sp - m_f), axis=1, keepdims=True)
        ptr_ref[...] = jnp.exp(sp - m_f) / l_f
        corr_ref[...] = jnp.exp(mh_scr[...] - m_f) / l_f


def _tc_stage(x, w_pad, b_pad, sp):
    NB = x.shape[0]
    BB = NB
    return pl.pallas_call(
        _tc_body,
        grid=(NB // BB, J),
        in_specs=[
            pl.BlockSpec((BB, D), lambda i, j: (i, 0)),
            pl.BlockSpec((D, BN), lambda i, j: (0, j)),
            pl.BlockSpec((1, BN), lambda i, j: (0, j)),
            pl.BlockSpec((BB, LPAD), lambda i, j: (i, 0)),
        ],
        out_specs=[
            pl.BlockSpec((BB, BN // 128, 128), lambda i, j: (i, j, 0)),
            pl.BlockSpec((BB, JPAD), lambda i, j: (i, 0)),
            pl.BlockSpec((BB, LPAD), lambda i, j: (i, 0)),
        ],
        out_shape=[
            jax.ShapeDtypeStruct((NB, GPAD // 128, 128), jnp.float32),
            jax.ShapeDtypeStruct((NB, JPAD), jnp.float32),
            jax.ShapeDtypeStruct((NB, LPAD), jnp.float32),
        ],
        scratch_shapes=[
            pltpu.VMEM((BB, 1), jnp.float32),
            pltpu.VMEM((BB, 1), jnp.float32),
            pltpu.VMEM((BB, JPAD), jnp.float32),
        ],
        compiler_params=pltpu.CompilerParams(
            dimension_semantics=("arbitrary", "arbitrary")),
    )(x, w_pad, b_pad, sp)


NW = 32
GRP = 4
CHP = 56
NPC = (GPAD // 128) // CHP
CHI = 3584
NIC = GPAD // CHI
IPP = (CHP * 128) // CHI
NVI = CHI // 16
SEC = OPAD // 4


def _sc_scatter(probs, corr, ptr, ctx, gen_to_out, inp_to_out):
    NB = ptr.shape[0]
    RPT = NB // NW
    mesh = plsc.VectorSubcoreMesh(core_axis_name="c", subcore_axis_name="s")

    @functools.partial(
        pl.kernel,
        out_type=jax.ShapeDtypeStruct((NB * OPAD,), jnp.float32),
        mesh=mesh,
        scratch_types=[
            pltpu.VMEM((OPAD,), jnp.float32),
            pltpu.VMEM((2, CHP, 128), jnp.float32),
            pltpu.VMEM((2, CHI), jnp.int32),
            pltpu.VMEM((GRP, JPAD), jnp.float32),
            pltpu.VMEM((GRP, LPAD), jnp.float32),
            pltpu.VMEM((GRP, LPAD), jnp.int32),
            pltpu.VMEM((13, 16), jnp.int32),
            pltpu.SemaphoreType.DMA,
            pltpu.SemaphoreType.DMA,
            pltpu.SemaphoreType.DMA,
            pltpu.SemaphoreType.DMA,
            pltpu.SemaphoreType.DMA,
            pltpu.SemaphoreType.DMA,
        ],
        compiler_params=pltpu.CompilerParams(needs_layout_passes=False),
    )
    def k(sraw_h, corr_h, ptr_h, ctx_h, gidx_h, ito_h, out_h,
          acc, pbuf, ibuf, corrbuf, ptrbuf, ctxbuf, ctxobuf,
          semp0, semp1, semi0, semi1, sem_c, sem_o):
        cid = lax.axis_index("c")
        sid = lax.axis_index("s")
        wid = sid * 2 + cid
        base = wid * RPT
        semp = (semp0, semp1)
        semi = (semi0, semi1)

        def zero_sec(s4):
            def zbody(i, carry):
                acc[pl.ds(s4 * SEC + i * 16, 16)] = jnp.zeros(
                    (16,), jnp.float32)
                return carry
            lax.fori_loop(0, SEC // 16, zbody, 0, unroll=16)

        for s4 in range(4):
            zero_sec(s4)

        def pchunk_copy(row, pc, bufsel):
            return pltpu.make_async_copy(
                sraw_h.at[row, pl.ds(pc * CHP, CHP)], pbuf.at[bufsel],
                semp[bufsel])

        def ichunk_copy(ic, bufsel):
            return pltpu.make_async_copy(
                gidx_h.at[pl.ds(ic * CHI, CHI)], ibuf.at[bufsel],
                semi[bufsel])

        def prime(row):
            for w in range(2):
                pchunk_copy(row, w, w).start()
                ichunk_copy(w, w).start()

        prime(base)

        def grp_body(g, carry):
            gbase = base + g * GRP
            pltpu.sync_copy(corr_h.at[pl.ds(gbase, GRP)], corrbuf)
            pltpu.sync_copy(ptr_h.at[pl.ds(gbase, GRP)], ptrbuf)
            pltpu.sync_copy(ctx_h.at[pl.ds(gbase, GRP)], ctxbuf)

            def row_body(rr, carry):
                row = gbase + rr
                gath = []
                for kk in range(13):
                    h = pltpu.make_async_copy(
                        ito_h.at[ctxbuf.at[rr, pl.ds(kk * 16, 16)]],
                        ctxobuf.at[kk], sem_c)
                    h.start()
                    gath.append(h)

                for pc in range(NPC):
                    pp = pc % 2
                    pchunk_copy(row, pc, pp).wait()
                    for ici in range(IPP):
                        ic = pc * IPP + ici
                        pi = ic % 2
                        ichunk_copy(ic, pi).wait()

                        def scat(kv, carry):
                            iv = ibuf[pi, pl.ds(kv * 16, 16)]
                            r = ici * (CHI // 128) + kv // 8
                            c = pl.multiple_of((kv % 8) * 16, 16)
                            pv = pbuf[pp, r, pl.ds(c, 16)]
                            jb = (ic * NVI + kv) // (BN // 16)
                            cv = plsc.load_gather(
                                corrbuf,
                                [jnp.full((16,), rr, jnp.int32),
                                 jnp.full((16,), jb, jnp.int32)])
                            plsc.addupdate_scatter(acc, [iv], pv * cv)
                            return carry

                        lax.fori_loop(0, NVI, scat, 0, unroll=16)
                        if ic + 2 < NIC:
                            ichunk_copy(ic + 2, pi).start()
                    if pc + 2 < NPC:
                        pchunk_copy(row, pc + 2, pp).start()

                for kk in range(13):
                    gath[kk].wait()
                for kk in range(13):
                    pv = ptrbuf[rr, pl.ds(kk * 16, 16)]
                    plsc.addupdate_scatter(acc, [ctxobuf[kk]], pv)

                @pl.when(row + 1 < base + RPT)
                def _():
                    prime(row + 1)

                secs = []
                for s4 in range(4):
                    h = pltpu.make_async_copy(
                        acc.at[pl.ds(s4 * SEC, SEC)],
                        out_h.at[pl.ds(row * OPAD + s4 * SEC, SEC)],
                        sem_o)
                    h.start()
                    secs.append(h)
                for s4 in range(4):
                    secs[s4].wait()
                    zero_sec(s4)
                return carry

            lax.fori_loop(0, GRP, row_body, 0)
            return carry

        lax.fori_loop(0, RPT // GRP, grp_body, 0)

    return k(probs, corr, ptr, ctx, gen_to_out, inp_to_out)


def kernel(x, scores, ctx_inp, W_gen, b_gen, gen_to_out, inp_to_out):
    w_pad = jnp.pad(W_gen, ((0, 0), (0, GPAD - GEN_VOC))).astype(jnp.bfloat16)
    b_pad = jnp.pad(b_gen, (0, GPAD - GEN_VOC)).reshape(1, GPAD)
    sp = jnp.pad(scores.astype(jnp.float32), ((0, 0), (0, LPAD - L)),
                 constant_values=NEG)
    ctxp = jnp.pad(ctx_inp.astype(jnp.int32), ((0, 0), (0, LPAD - L)))
    gidx = jnp.pad(gen_to_out.astype(jnp.int32), (0, GPAD - GEN_VOC))
    ito = inp_to_out.astype(jnp.int32)
    xb = x.astype(jnp.bfloat16)

    NH = 1
    NBH = B // NH
    outs = []
    for h in range(NH):
        rs = slice(h * NBH, (h + 1) * NBH)
        probs, corr, ptr = _tc_stage(xb[rs], w_pad, b_pad, sp[rs])
        out_flat = _sc_scatter(probs, corr, ptr, ctxp[rs], gidx, ito)
        outs.append(out_flat.reshape(NBH, OPAD)[:, :OUT_VOC])
    return jnp.concatenate(outs, axis=0)


# speedup vs baseline: 1.1171x; 1.0359x over previous
import functools

import jax
import jax.numpy as jnp
from jax import lax
from jax.experimental import pallas as pl
from jax.experimental.pallas import tpu as pltpu
from jax.experimental.pallas import tpu_sc as plsc

B = 1024
D = 256
GEN_VOC = 50000
OUT_VOC = 100000
L = 200

GPAD = 50176
BN = 1024
J = GPAD // BN
BB = 1024
JPAD = 128
LPAD = 256
OPAD = 100096

NEG = -1e30


def _tc_body(x_ref, w_ref, b_ref, sp_ref, sraw_ref, corr_ref, ptr_ref,
             m_scr, l_scr, mh_scr):
    j = pl.program_id(1)
    BB = x_ref.shape[0]
    s = jnp.dot(x_ref[...], w_ref[...], preferred_element_type=jnp.float32)
    s = s + b_ref[...]

    col = j * BN + lax.broadcasted_iota(jnp.int32, s.shape, 1)
    sm = jnp.where(col < GEN_VOC, s, NEG)

    @pl.when(j == 0)
    def _():
        m_scr[...] = jnp.full((BB, 1), NEG, jnp.float32)
        l_scr[...] = jnp.zeros((BB, 1), jnp.float32)
        mh_scr[...] = jnp.full((BB, JPAD), NEG, jnp.float32)

    m_old = m_scr[...]
    l_old = l_scr[...]
    pm = jnp.max(sm, axis=1, keepdims=True)
    m_new = jnp.maximum(m_old, pm)
    l_new = l_old * jnp.exp(m_old - m_new) + jnp.sum(
        jnp.exp(sm - m_new), axis=1, keepdims=True)
    m_scr[...] = m_new
    l_scr[...] = l_new
    jcol = lax.broadcasted_iota(jnp.int32, (BB, JPAD), 1)
    mh_scr[...] = jnp.where(jcol == j, m_new, mh_scr[...])

    sraw_ref[...] = jnp.exp(sm - m_new).reshape(BB, BN // 128, 128)

    @pl.when(j == J - 1)
    def _():
        sp = sp_ref[...]
        pm2 = jnp.max(sp, axis=1, keepdims=True)
        m_f = jnp.maximum(m_new, pm2)
        l_f = l_new * jnp.exp(m_new - m_f) + jnp.sum(
            jnp.exp(sp - m_f), axis=1, keepdims=True)
        ptr_ref[...] = jnp.exp(sp - m_f) / l_f
        corr_ref[...] = jnp.exp(mh_scr[...] - m_f) / l_f


def _tc_stage(x, w_pad, b_pad, sp):
    NB = x.shape[0]
    BB = NB
    return pl.pallas_call(
        _tc_body,
        grid=(NB // BB, J),
        in_specs=[
            pl.BlockSpec((BB, D), lambda i, j: (i, 0)),
            pl.BlockSpec((D, BN), lambda i, j: (0, j)),
            pl.BlockSpec((1, BN), lambda i, j: (0, j)),
            pl.BlockSpec((BB, LPAD), lambda i, j: (i, 0)),
        ],
        out_specs=[
            pl.BlockSpec((BB, BN // 128, 128), lambda i, j: (i, j, 0)),
            pl.BlockSpec((BB, JPAD), lambda i, j: (i, 0)),
            pl.BlockSpec((BB, LPAD), lambda i, j: (i, 0)),
        ],
        out_shape=[
            jax.ShapeDtypeStruct((NB, GPAD // 128, 128), jnp.float32),
            jax.ShapeDtypeStruct((NB, JPAD), jnp.float32),
            jax.ShapeDtypeStruct((NB, LPAD), jnp.float32),
        ],
        scratch_shapes=[
            pltpu.VMEM((BB, 1), jnp.float32),
            pltpu.VMEM((BB, 1), jnp.float32),
            pltpu.VMEM((BB, JPAD), jnp.float32),
        ],
        compiler_params=pltpu.CompilerParams(
            dimension_semantics=("arbitrary", "arbitrary")),
    )(x, w_pad, b_pad, sp)


NW = 32
GRP = 4
CHP = 56
NPC = (GPAD // 128) // CHP
CHI = 3584
NIC = GPAD // CHI
IPP = (CHP * 128) // CHI
NVI = CHI // 16
SEC = OPAD // 4


def _sc_scatter(probs, corr, ptr, ctx, gen_to_out, inp_to_out):
    NB = ptr.shape[0]
    RPT = NB // NW
    mesh = plsc.VectorSubcoreMesh(core_axis_name="c", subcore_axis_name="s")

    @functools.partial(
        pl.kernel,
        out_type=jax.ShapeDtypeStruct((NB * OPAD,), jnp.float32),
        mesh=mesh,
        scratch_types=[
            pltpu.VMEM((OPAD,), jnp.float32),
            pltpu.VMEM((2, CHP, 128), jnp.float32),
            pltpu.VMEM((2, CHI), jnp.int32),
            pltpu.VMEM((GRP, JPAD), jnp.float32),
            pltpu.VMEM((GRP, LPAD), jnp.float32),
            pltpu.VMEM((GRP, LPAD), jnp.int32),
            pltpu.VMEM((13, 16), jnp.int32),
            pltpu.VMEM_SHARED((GPAD,), jnp.int32),
            pltpu.SemaphoreType.DMA,
            pltpu.SemaphoreType.DMA,
            pltpu.SemaphoreType.DMA,
            pltpu.SemaphoreType.DMA,
            pltpu.SemaphoreType.DMA,
            pltpu.SemaphoreType.DMA,
        ],
        compiler_params=pltpu.CompilerParams(needs_layout_passes=False),
    )
    def k(sraw_h, corr_h, ptr_h, ctx_h, gidx_h, ito_h, out_h,
          acc, pbuf, ibuf, corrbuf, ptrbuf, ctxbuf, ctxobuf, gshared,
          semp0, semp1, semi0, semi1, sem_c, sem_o):
        cid = lax.axis_index("c")
        sid = lax.axis_index("s")
        wid = sid * 2 + cid
        base = wid * RPT
        semp = (semp0, semp1)
        semi = (semi0, semi1)

        def zero_sec(s4):
            def zbody(i, carry):
                acc[pl.ds(s4 * SEC + i * 16, 16)] = jnp.zeros(
                    (16,), jnp.float32)
                return carry
            lax.fori_loop(0, SEC // 16, zbody, 0, unroll=16)

        for s4 in range(4):
            zero_sec(s4)

        def pchunk_copy(row, pc, bufsel):
            return pltpu.make_async_copy(
                sraw_h.at[row, pl.ds(pc * CHP, CHP)], pbuf.at[bufsel],
                semp[bufsel])

        @pl.when(sid == 0)
        def _():
            pltpu.sync_copy(gidx_h, gshared)
        plsc.subcore_barrier()

        def ichunk_copy(ic, bufsel):
            return pltpu.make_async_copy(
                gshared.at[pl.ds(ic * CHI, CHI)], ibuf.at[bufsel],
                semi[bufsel])

        def prime(row):
            for w in range(2):
                pchunk_copy(row, w, w).start()
                ichunk_copy(w, w).start()

        prime(base)

        def grp_body(g, carry):
            gbase = base + g * GRP
            pltpu.sync_copy(corr_h.at[pl.ds(gbase, GRP)], corrbuf)
            pltpu.sync_copy(ptr_h.at[pl.ds(gbase, GRP)], ptrbuf)
            pltpu.sync_copy(ctx_h.at[pl.ds(gbase, GRP)], ctxbuf)

            def row_body(rr, carry):
                row = gbase + rr
                gath = []
                for kk in range(13):
                    h = pltpu.make_async_copy(
                        ito_h.at[ctxbuf.at[rr, pl.ds(kk * 16, 16)]],
                        ctxobuf.at[kk], sem_c)
                    h.start()
                    gath.append(h)

                for pc in range(NPC):
                    pp = pc % 2
                    pchunk_copy(row, pc, pp).wait()
                    for ici in range(IPP):
                        ic = pc * IPP + ici
                        pi = ic % 2
                        ichunk_copy(ic, pi).wait()

                        def scat(kv, carry):
                            iv = ibuf[pi, pl.ds(kv * 16, 16)]
                            r = ici * (CHI // 128) + kv // 8
                            c = pl.multiple_of((kv % 8) * 16, 16)
                            pv = pbuf[pp, r, pl.ds(c, 16)]
                            jb = (ic * NVI + kv) // (BN // 16)
                            cv = plsc.load_gather(
                                corrbuf,
                                [jnp.full((16,), rr, jnp.int32),
                                 jnp.full((16,), jb, jnp.int32)])
                            plsc.addupdate_scatter(acc, [iv], pv * cv)
                            return carry

                        lax.fori_loop(0, NVI, scat, 0, unroll=16)
                        if ic + 2 < NIC:
                            ichunk_copy(ic + 2, pi).start()
                    if pc + 2 < NPC:
                        pchunk_copy(row, pc + 2, pp).start()

                for kk in range(13):
                    gath[kk].wait()
                for kk in range(13):
                    pv = ptrbuf[rr, pl.ds(kk * 16, 16)]
                    plsc.addupdate_scatter(acc, [ctxobuf[kk]], pv)

                @pl.when(row + 1 < base + RPT)
                def _():
                    prime(row + 1)

                secs = []
                for s4 in range(4):
                    h = pltpu.make_async_copy(
                        acc.at[pl.ds(s4 * SEC, SEC)],
                        out_h.at[pl.ds(row * OPAD + s4 * SEC, SEC)],
                        sem_o)
                    h.start()
                    secs.append(h)
                for s4 in range(4):
                    secs[s4].wait()
                    zero_sec(s4)
                return carry

            lax.fori_loop(0, GRP, row_body, 0)
            return carry

        lax.fori_loop(0, RPT // GRP, grp_body, 0)

    return k(probs, corr, ptr, ctx, gen_to_out, inp_to_out)


def kernel(x, scores, ctx_inp, W_gen, b_gen, gen_to_out, inp_to_out):
    w_pad = jnp.pad(W_gen, ((0, 0), (0, GPAD - GEN_VOC))).astype(jnp.bfloat16)
    b_pad = jnp.pad(b_gen, (0, GPAD - GEN_VOC)).reshape(1, GPAD)
    sp = jnp.pad(scores.astype(jnp.float32), ((0, 0), (0, LPAD - L)),
                 constant_values=NEG)
    ctxp = jnp.pad(ctx_inp.astype(jnp.int32), ((0, 0), (0, LPAD - L)))
    gidx = jnp.pad(gen_to_out.astype(jnp.int32), (0, GPAD - GEN_VOC))
    ito = inp_to_out.astype(jnp.int32)
    xb = x.astype(jnp.bfloat16)

    NH = 1
    NBH = B // NH
    outs = []
    for h in range(NH):
        rs = slice(h * NBH, (h + 1) * NBH)
        probs, corr, ptr = _tc_stage(xb[rs], w_pad, b_pad, sp[rs])
        out_flat = _sc_scatter(probs, corr, ptr, ctxp[rs], gidx, ito)
        outs.append(out_flat.reshape(NBH, OPAD)[:, :OUT_VOC])
    return jnp.concatenate(outs, axis=0)
